# baseline (speedup 1.0000x reference)
import functools
import jax
import jax.numpy as jnp
from jax import lax
from jax.experimental import pallas as pl
from jax.experimental.pallas import tpu as pltpu
from jax.experimental.pallas import tpu_sc as plsc

N = 10000
E = 320000
NC, NS = 2, 16
NW = NC * NS
EPW = E // NW
CB = 80
GSZ = 5
NG = EPW // (CB * GSZ)
NP = 10240
RPT = NP // NS
ZR = 128
HD = 64
NBLK = 10
BLK = N // NBLK

_mesh = plsc.VectorSubcoreMesh(core_axis_name="c", subcore_axis_name="s")


def _ds8(off, size):
    return pl.ds(pl.multiple_of(off, 8), size)


@functools.partial(
    pl.kernel,
    out_type=jax.ShapeDtypeStruct((NC, NP, 16), jnp.float32),
    mesh=_mesh,
    compiler_params=pltpu.CompilerParams(use_tc_tiling_on_sc=False),
    scratch_types=[
        [pltpu.VMEM((CB,), jnp.int32) for _ in range(2 * GSZ)],
        [pltpu.VMEM((CB,), jnp.int32) for _ in range(2 * GSZ)],
        pltpu.VMEM((CB, 16), jnp.float32),
        pltpu.VMEM((CB, 16), jnp.float32),
        pltpu.VMEM((ZR, 16), jnp.float32),
        pltpu.VMEM_SHARED((NP, 16), jnp.float32),
        pltpu.SemaphoreType.DMA,
        pltpu.SemaphoreType.DMA,
        pltpu.SemaphoreType.DMA,
        pltpu.SemaphoreType.DMA,
    ],
)
def _sc_degrees(src_hbm, dst_hbm, out_hbm, sidx, didx, ones0, ones1, zbuf,
                acc, st0, st1, ss0, ss1):
    c = lax.axis_index("c")
    s = lax.axis_index("s")
    w = c * NS + s
    ebase = w * EPW
    stsem = (st0, st1)
    ssem = (ss0, ss1)

    lane = lax.iota(jnp.int32, 16)
    o0 = jnp.where(lane == 0, 1.0, 0.0).astype(jnp.float32)
    o1 = jnp.where(lane == 1, 1.0, 0.0).astype(jnp.float32)
    zv = jnp.zeros((16,), jnp.float32)

    def fill(i, carry):
        ones0[i, pl.ds(0, 16)] = o0
        ones1[i, pl.ds(0, 16)] = o1
        return carry

    lax.fori_loop(0, CB, fill, 0)

    def zfill(i, carry):
        zbuf[i, pl.ds(0, 16)] = zv
        return carry

    lax.fori_loop(0, ZR, zfill, 0)

    base = s * RPT
    for k in range(RPT // ZR):
        pltpu.sync_copy(zbuf, acc.at[pl.ds(base + k * ZR, ZR)])
    plsc.subcore_barrier()

    def stage(g, p):
        for i in range(GSZ):
            off = ebase + (g * GSZ + i) * CB
            pltpu.async_copy(src_hbm.at[_ds8(off, CB)], sidx[p * GSZ + i],
                             stsem[p])
            pltpu.async_copy(dst_hbm.at[_ds8(off, CB)], didx[p * GSZ + i],
                             stsem[p])

    def drain_stage(g, p):
        for i in range(GSZ):
            off = ebase + (g * GSZ + i) * CB
            pltpu.make_async_copy(src_hbm.at[_ds8(off, CB)],
                                  sidx[p * GSZ + i], stsem[p]).wait()
            pltpu.make_async_copy(dst_hbm.at[_ds8(off, CB)],
                                  didx[p * GSZ + i], stsem[p]).wait()

    def fire_scatters(p):
        for i in range(GSZ):
            pltpu.async_copy(ones0, acc.at[sidx[p * GSZ + i]], ssem[p],
                             add=True)
            pltpu.async_copy(ones1, acc.at[didx[p * GSZ + i]], ssem[p],
                             add=True)

    def drain_scatters(p):
        for i in range(GSZ):
            pltpu.make_async_copy(ones0, acc.at[sidx[p * GSZ + i]],
                                  ssem[p]).wait()
            pltpu.make_async_copy(ones1, acc.at[didx[p * GSZ + i]],
                                  ssem[p]).wait()

    def body(g, p, first, last):
        q = 1 - p
        if not first:
            drain_scatters(q)
        if not last:
            stage(g + 1, q)
        drain_stage(g, p)
        fire_scatters(p)

    stage(0, 0)
    body(0, 0, True, False)

    def pair(k, carry):
        g0 = k * 2 + 1
        body(g0, 1, False, False)
        body(g0 + 1, 0, False, False)
        return carry

    lax.fori_loop(0, (NG - 3) // 2, pair, 0)
    body(NG - 2, (NG - 2) % 2, False, False)
    body(NG - 1, (NG - 1) % 2, False, True)
    drain_scatters((NG - 1) % 2)

    plsc.subcore_barrier()
    pltpu.sync_copy(acc.at[pl.ds(base, RPT)], out_hbm.at[c, pl.ds(base, RPT)])


def _make_sc_scatter(nh):
    @functools.partial(
        pl.kernel,
        out_type=jax.ShapeDtypeStruct((nh, NC, NP, HD), jnp.float32),
        mesh=_mesh,
        compiler_params=pltpu.CompilerParams(use_tc_tiling_on_sc=False),
        scratch_types=[
            pltpu.VMEM((EPW,), jnp.int32),
            [pltpu.VMEM((CB,), jnp.int32) for _ in range(2 * GSZ)],
            [pltpu.VMEM((CB, HD), jnp.float32) for _ in range(2 * GSZ)],
            pltpu.VMEM((ZR, HD), jnp.float32),
            pltpu.VMEM_SHARED((NP, HD), jnp.float32),
            [pltpu.SemaphoreType.DMA for _ in range(6)],
        ],
    )
    def k(*refs):
        ys = refs[:nh]
        src_hbm, dst_hbm, out_hbm = refs[nh:nh + 3]
        (srcv, didx, dbuf, zbuf, acc, sems) = refs[nh + 3:]
        c = lax.axis_index("c")
        s = lax.axis_index("s")
        w = c * NS + s
        ebase = w * EPW
        stsem = sems[0:2]
        gsem = sems[2:4]
        ssem = sems[4:6]

        zv = jnp.zeros((16,), jnp.float32)

        def zfill(i, carry):
            for g in range(HD // 16):
                zbuf[i, pl.ds(g * 16, 16)] = zv
            return carry

        lax.fori_loop(0, ZR, zfill, 0)

        pltpu.sync_copy(src_hbm.at[_ds8(ebase, EPW)], srcv)
        base = s * RPT

        def stage(g, p):
            for i in range(GSZ):
                off = ebase + (g * GSZ + i) * CB
                pltpu.async_copy(dst_hbm.at[_ds8(off, CB)],
                                 didx[p * GSZ + i], stsem[p])

        def drain_stage(g, p):
            for i in range(GSZ):
                off = ebase + (g * GSZ + i) * CB
                pltpu.make_async_copy(dst_hbm.at[_ds8(off, CB)],
                                      didx[p * GSZ + i], stsem[p]).wait()

        def fire_gathers(y_hbm, g, p):
            for i in range(GSZ):
                off = (g * GSZ + i) * CB
                pltpu.async_copy(y_hbm.at[srcv.at[_ds8(off, CB)]],
                                 dbuf[p * GSZ + i], gsem[p])

        def drain_gathers(y_hbm, g, p):
            for i in range(GSZ):
                off = (g * GSZ + i) * CB
                pltpu.make_async_copy(y_hbm.at[srcv.at[_ds8(off, CB)]],
                                      dbuf[p * GSZ + i], gsem[p]).wait()

        def fire_scatters(p):
            for i in range(GSZ):
                pltpu.async_copy(dbuf[p * GSZ + i], acc.at[didx[p * GSZ + i]],
                                 ssem[p], add=True)

        def drain_scatters(p):
            for i in range(GSZ):
                pltpu.make_async_copy(dbuf[p * GSZ + i],
                                      acc.at[didx[p * GSZ + i]],
                                      ssem[p]).wait()

        for half in range(nh):
            y_hbm = ys[half]
            for k2 in range(RPT // ZR):
                pltpu.sync_copy(zbuf, acc.at[pl.ds(base + k2 * ZR, ZR)])
            plsc.subcore_barrier()

            def body(g, p, first, last):
                q = 1 - p
                drain_gathers(y_hbm, g, p)
                drain_stage(g, p)
                fire_scatters(p)
                if not first:
                    drain_scatters(q)
                if not last:
                    stage(g + 1, q)
                    fire_gathers(y_hbm, g + 1, q)

            stage(0, 0)
            fire_gathers(y_hbm, 0, 0)
            body(0, 0, True, False)

            def pair(k3, carry):
                g0 = k3 * 2 + 1
                body(g0, 1, False, False)
                body(g0 + 1, 0, False, False)
                return carry

            lax.fori_loop(0, (NG - 3) // 2, pair, 0)
            body(NG - 2, (NG - 2) % 2, False, False)
            body(NG - 1, (NG - 1) % 2, False, True)
            drain_scatters((NG - 1) % 2)

            plsc.subcore_barrier()
            pltpu.sync_copy(acc.at[pl.ds(base, RPT)],
                            out_hbm.at[half, c, pl.ds(base, RPT)])
            plsc.subcore_barrier()

    return k


_sc_scatter_h2 = _make_sc_scatter(2)
_sc_scatter_h1 = _make_sc_scatter(1)


def _tc_ab_body(hist_ref, a_ref, b_ref):
    h = hist_ref[...]
    dego = jnp.maximum(h[0, :, 0:1] + h[1, :, 0:1], 1.0)
    degi = jnp.maximum(h[0, :, 1:2] + h[1, :, 1:2], 1.0)
    a_ref[...] = lax.rsqrt(dego)
    b_ref[...] = lax.rsqrt(degi)


def _tc_ab(hist):
    return pl.pallas_call(
        _tc_ab_body,
        grid=(NBLK,),
        in_specs=[pl.BlockSpec((NC, BLK, 16), lambda i: (0, i, 0))],
        out_specs=[
            pl.BlockSpec((BLK, 1), lambda i: (i, 0)),
            pl.BlockSpec((BLK, 1), lambda i: (i, 0)),
        ],
        out_shape=[
            jax.ShapeDtypeStruct((N, 1), jnp.float32),
            jax.ShapeDtypeStruct((N, 1), jnp.float32),
        ],
    )(hist)


def _tc_mm_body(nh, x_ref, a_ref, w0_ref, w1_ref, z_ref, *y_refs):
    x = x_ref[...]
    z_ref[...] = jnp.dot(x, w0_ref[...], preferred_element_type=jnp.float32)
    y = jnp.dot(x * a_ref[...], w1_ref[...],
                preferred_element_type=jnp.float32)
    for h in range(nh):
        y_refs[h][...] = y[:, h * HD:(h + 1) * HD]


def _tc_mm(x, a, w0, w1, nh):
    D = nh * HD
    return pl.pallas_call(
        functools.partial(_tc_mm_body, nh),
        grid=(NBLK,),
        in_specs=[
            pl.BlockSpec((BLK, 128), lambda i: (i, 0)),
            pl.BlockSpec((BLK, 1), lambda i: (i, 0)),
            pl.BlockSpec((128, D), lambda i: (0, 0)),
            pl.BlockSpec((128, D), lambda i: (0, 0)),
        ],
        out_specs=[pl.BlockSpec((BLK, D), lambda i: (i, 0))] +
        [pl.BlockSpec((BLK, HD), lambda i: (i, 0)) for _ in range(nh)],
        out_shape=[jax.ShapeDtypeStruct((N, D), jnp.float32)] +
        [jax.ShapeDtypeStruct((N, HD), jnp.float32) for _ in range(nh)],
    )(x, a, w0, w1)


def _tc_pre_body(nh, z_ref, s_ref, b_ref, sn_ref, bias_ref, pre_ref, st_ref,
                 acc):
    i = pl.program_id(0)

    @pl.when(i == 0)
    def _():
        acc[...] = jnp.zeros_like(acc)

    sv = s_ref[...]
    halves = [sv[h, 0] + sv[h, 1] for h in range(nh)]
    s_full = halves[0] if nh == 1 else jnp.concatenate(halves, axis=-1)
    pre = (z_ref[...] - b_ref[...] * s_full + bias_ref[...])
    pre = pre * sn_ref[...]
    pre_ref[...] = pre
    acc[0:1, :] += jnp.sum(pre, axis=0, keepdims=True)
    acc[1:2, :] += jnp.sum(pre * pre, axis=0, keepdims=True)
    st_ref[...] = acc[...]


def _tc_pre(z, s, b, snorm, bias, nh):
    D = nh * HD
    return pl.pallas_call(
        functools.partial(_tc_pre_body, nh),
        grid=(NBLK,),
        in_specs=[
            pl.BlockSpec((BLK, D), lambda i: (i, 0)),
            pl.BlockSpec((nh, NC, BLK, HD), lambda i: (0, 0, i, 0)),
            pl.BlockSpec((BLK, 1), lambda i: (i, 0)),
            pl.BlockSpec((BLK, 1), lambda i: (i, 0)),
            pl.BlockSpec((1, D), lambda i: (0, 0)),
        ],
        out_specs=[
            pl.BlockSpec((BLK, D), lambda i: (i, 0)),
            pl.BlockSpec((8, D), lambda i: (0, 0)),
        ],
        out_shape=[
            jax.ShapeDtypeStruct((N, D), jnp.float32),
            jax.ShapeDtypeStruct((8, D), jnp.float32),
        ],
        scratch_shapes=[pltpu.VMEM((8, D), jnp.float32)],
    )(z, s, b, snorm, bias)


def _tc_norm_body(pre_ref, st_ref, g_ref, be_ref, xin_ref, o_ref, *, relu,
                  resid):
    st = st_ref[...]
    mean = st[0:1, :] * (1.0 / N)
    var = st[1:2, :] * (1.0 / N) - mean * mean
    inv = lax.rsqrt(var + 1e-5)
    o = (pre_ref[...] - mean) * inv * g_ref[...] + be_ref[...]
    if relu:
        o = jnp.maximum(o, 0.0)
    if resid:
        o = o + xin_ref[...]
    o_ref[...] = o


def _tc_norm(pre, st, gamma, beta, xin, D, relu, resid):
    return pl.pallas_call(
        functools.partial(_tc_norm_body, relu=relu, resid=resid),
        grid=(NBLK,),
        in_specs=[
            pl.BlockSpec((BLK, D), lambda i: (i, 0)),
            pl.BlockSpec((8, D), lambda i: (0, 0)),
            pl.BlockSpec((1, D), lambda i: (0, 0)),
            pl.BlockSpec((1, D), lambda i: (0, 0)),
            pl.BlockSpec((BLK, 128), lambda i: (i, 0)),
        ],
        out_specs=pl.BlockSpec((BLK, D), lambda i: (i, 0)),
        out_shape=jax.ShapeDtypeStruct((N, D), jnp.float32),
    )(pre, st, gamma, beta, xin)


def kernel(h, edge_index, e, snorm_n, snorm_e,
           W0_l0, W1_l0, b_l0, gamma_l0, beta_l0,
           W0_l1, W1_l1, b_l1, gamma_l1, beta_l1,
           W0_l2, W1_l2, b_l2, gamma_l2, beta_l2,
           W0_l3, W1_l3, b_l3, gamma_l3, beta_l3,
           W0_l4, W1_l4, b_l4, gamma_l4, beta_l4):
    srcf = edge_index[0]
    dstf = edge_index[1]

    hist = _sc_degrees(srcf, dstf)
    a, b = _tc_ab(hist)

    layers = [
        (W0_l0, W1_l0, b_l0, gamma_l0, beta_l0),
        (W0_l1, W1_l1, b_l1, gamma_l1, beta_l1),
        (W0_l2, W1_l2, b_l2, gamma_l2, beta_l2),
        (W0_l3, W1_l3, b_l3, gamma_l3, beta_l3),
        (W0_l4, W1_l4, b_l4, gamma_l4, beta_l4),
    ]

    x = h
    for li, (w0, w1, bias, gamma, beta) in enumerate(layers):
        last = li == len(layers) - 1
        nh = 1 if last else 2
        D = nh * HD
        if last:
            w0 = jnp.pad(w0, ((0, 0), (0, D - w0.shape[1])))
            w1 = jnp.pad(w1, ((0, 0), (0, D - w1.shape[1])))
            bias = jnp.pad(bias, (0, D - bias.shape[0]))
            gamma = jnp.pad(gamma, (0, D - gamma.shape[0]))
            beta = jnp.pad(beta, (0, D - beta.shape[0]))
        outs = _tc_mm(x, a, w0, w1, nh)
        z, ys = outs[0], outs[1:]
        if last:
            s = _sc_scatter_h1(ys[0], srcf, dstf)
        else:
            s = _sc_scatter_h2(ys[0], ys[1], srcf, dstf)
        pre, st = _tc_pre(z, s, b, snorm_n, bias.reshape(1, D), nh)
        x = _tc_norm(pre, st, gamma.reshape(1, D), beta.reshape(1, D), x, D,
                     relu=not last, resid=not last)
    return x[:, :40]


# speedup vs baseline: 1.0578x; 1.0578x over previous
import functools
import jax
import jax.numpy as jnp
from jax import lax
from jax.experimental import pallas as pl
from jax.experimental.pallas import tpu as pltpu
from jax.experimental.pallas import tpu_sc as plsc

N = 10000
E = 320000
NC, NS = 2, 16
NW = NC * NS
EPW = E // NW
CB = 80
GSZ = 5
NG = EPW // (CB * GSZ)
NP = 10240
RPT = NP // NS
ZR = 128
HD = 64
NBLK = 10
BLK = N // NBLK

_mesh = plsc.VectorSubcoreMesh(core_axis_name="c", subcore_axis_name="s")


def _ds8(off, size):
    return pl.ds(pl.multiple_of(off, 8), size)


@functools.partial(
    pl.kernel,
    out_type=jax.ShapeDtypeStruct((NC, NP, 16), jnp.float32),
    mesh=_mesh,
    compiler_params=pltpu.CompilerParams(use_tc_tiling_on_sc=False),
    scratch_types=[
        [pltpu.VMEM((CB,), jnp.int32) for _ in range(2 * GSZ)],
        [pltpu.VMEM((CB,), jnp.int32) for _ in range(2 * GSZ)],
        pltpu.VMEM((CB, 16), jnp.float32),
        pltpu.VMEM((CB, 16), jnp.float32),
        pltpu.VMEM((ZR, 16), jnp.float32),
        pltpu.VMEM_SHARED((NP, 16), jnp.float32),
        pltpu.SemaphoreType.DMA,
        pltpu.SemaphoreType.DMA,
        pltpu.SemaphoreType.DMA,
        pltpu.SemaphoreType.DMA,
    ],
)
def _sc_degrees(src_hbm, dst_hbm, out_hbm, sidx, didx, ones0, ones1, zbuf,
                acc, st0, st1, ss0, ss1):
    c = lax.axis_index("c")
    s = lax.axis_index("s")
    w = c * NS + s
    ebase = w * EPW
    stsem = (st0, st1)
    ssem = (ss0, ss1)

    lane = lax.iota(jnp.int32, 16)
    o0 = jnp.where(lane == 0, 1.0, 0.0).astype(jnp.float32)
    o1 = jnp.where(lane == 1, 1.0, 0.0).astype(jnp.float32)
    zv = jnp.zeros((16,), jnp.float32)

    def fill(i, carry):
        ones0[i, pl.ds(0, 16)] = o0
        ones1[i, pl.ds(0, 16)] = o1
        return carry

    lax.fori_loop(0, CB, fill, 0)

    def zfill(i, carry):
        zbuf[i, pl.ds(0, 16)] = zv
        return carry

    lax.fori_loop(0, ZR, zfill, 0)

    base = s * RPT
    for k in range(RPT // ZR):
        pltpu.sync_copy(zbuf, acc.at[pl.ds(base + k * ZR, ZR)])
    plsc.subcore_barrier()

    def stage(g, p):
        for i in range(GSZ):
            off = ebase + (g * GSZ + i) * CB
            pltpu.async_copy(src_hbm.at[_ds8(off, CB)], sidx[p * GSZ + i],
                             stsem[p])
            pltpu.async_copy(dst_hbm.at[_ds8(off, CB)], didx[p * GSZ + i],
                             stsem[p])

    def drain_stage(g, p):
        for i in range(GSZ):
            off = ebase + (g * GSZ + i) * CB
            pltpu.make_async_copy(src_hbm.at[_ds8(off, CB)],
                                  sidx[p * GSZ + i], stsem[p]).wait()
            pltpu.make_async_copy(dst_hbm.at[_ds8(off, CB)],
                                  didx[p * GSZ + i], stsem[p]).wait()

    def fire_scatters(p):
        for i in range(GSZ):
            pltpu.async_copy(ones0, acc.at[sidx[p * GSZ + i]], ssem[p],
                             add=True)
            pltpu.async_copy(ones1, acc.at[didx[p * GSZ + i]], ssem[p],
                             add=True)

    def drain_scatters(p):
        for i in range(GSZ):
            pltpu.make_async_copy(ones0, acc.at[sidx[p * GSZ + i]],
                                  ssem[p]).wait()
            pltpu.make_async_copy(ones1, acc.at[didx[p * GSZ + i]],
                                  ssem[p]).wait()

    def body(g, p, first, last):
        q = 1 - p
        if not first:
            drain_scatters(q)
        if not last:
            stage(g + 1, q)
        drain_stage(g, p)
        fire_scatters(p)

    stage(0, 0)
    body(0, 0, True, False)

    def pair(k, carry):
        g0 = k * 2 + 1
        body(g0, 1, False, False)
        body(g0 + 1, 0, False, False)
        return carry

    lax.fori_loop(0, (NG - 3) // 2, pair, 0)
    body(NG - 2, (NG - 2) % 2, False, False)
    body(NG - 1, (NG - 1) % 2, False, True)
    drain_scatters((NG - 1) % 2)

    plsc.subcore_barrier()
    pltpu.sync_copy(acc.at[pl.ds(base, RPT)], out_hbm.at[c, pl.ds(base, RPT)])


def _make_sc_scatter(nh):
    @functools.partial(
        pl.kernel,
        out_type=jax.ShapeDtypeStruct((nh, NC, NP, HD), jnp.float32),
        mesh=_mesh,
        compiler_params=pltpu.CompilerParams(use_tc_tiling_on_sc=False),
        scratch_types=[
            pltpu.VMEM((EPW,), jnp.int32),
            [pltpu.VMEM((CB,), jnp.int32) for _ in range(2 * GSZ)],
            [pltpu.VMEM((CB, HD), jnp.float32) for _ in range(2 * GSZ)],
            pltpu.VMEM((ZR, HD), jnp.float32),
            pltpu.VMEM_SHARED((NP, HD), jnp.float32),
            pltpu.SemaphoreType.DMA,
            pltpu.SemaphoreType.DMA,
            pltpu.SemaphoreType.DMA,
            pltpu.SemaphoreType.DMA,
            pltpu.SemaphoreType.DMA,
            pltpu.SemaphoreType.DMA,
        ],
    )
    def k(*refs):
        ys = refs[:nh]
        src_hbm, dst_hbm, out_hbm = refs[nh:nh + 3]
        (srcv, didx, dbuf, zbuf, acc, st0, st1, sg0, sg1, ss0, ss1) = \
            refs[nh + 3:]
        c = lax.axis_index("c")
        s = lax.axis_index("s")
        w = c * NS + s
        ebase = w * EPW
        stsem = (st0, st1)
        gsem = (sg0, sg1)
        ssem = (ss0, ss1)

        zv = jnp.zeros((16,), jnp.float32)

        def zfill(i, carry):
            for g in range(HD // 16):
                zbuf[i, pl.ds(g * 16, 16)] = zv
            return carry

        lax.fori_loop(0, ZR, zfill, 0)

        pltpu.sync_copy(src_hbm.at[_ds8(ebase, EPW)], srcv)
        base = s * RPT

        def stage(g, p):
            for i in range(GSZ):
                off = ebase + (g * GSZ + i) * CB
                pltpu.async_copy(dst_hbm.at[_ds8(off, CB)],
                                 didx[p * GSZ + i], stsem[p])

        def drain_stage(g, p):
            for i in range(GSZ):
                off = ebase + (g * GSZ + i) * CB
                pltpu.make_async_copy(dst_hbm.at[_ds8(off, CB)],
                                      didx[p * GSZ + i], stsem[p]).wait()

        def fire_gathers(y_hbm, g, p):
            for i in range(GSZ):
                off = (g * GSZ + i) * CB
                pltpu.async_copy(y_hbm.at[srcv.at[_ds8(off, CB)]],
                                 dbuf[p * GSZ + i], gsem[p])

        def drain_gathers(y_hbm, g, p):
            for i in range(GSZ):
                off = (g * GSZ + i) * CB
                pltpu.make_async_copy(y_hbm.at[srcv.at[_ds8(off, CB)]],
                                      dbuf[p * GSZ + i], gsem[p]).wait()

        def fire_scatters(p):
            for i in range(GSZ):
                pltpu.async_copy(dbuf[p * GSZ + i], acc.at[didx[p * GSZ + i]],
                                 ssem[p], add=True)

        def drain_scatters(p):
            for i in range(GSZ):
                pltpu.make_async_copy(dbuf[p * GSZ + i],
                                      acc.at[didx[p * GSZ + i]],
                                      ssem[p]).wait()

        for half in range(nh):
            y_hbm = ys[half]
            do_stage = True
            for k2 in range(RPT // ZR):
                pltpu.sync_copy(zbuf, acc.at[pl.ds(base + k2 * ZR, ZR)])
            plsc.subcore_barrier()

            def body(g, p, first, last):
                q = 1 - p
                if not first:
                    drain_scatters(q)
                if do_stage and not last:
                    stage(g + 1, q)
                if not last:
                    fire_gathers(y_hbm, g + 1, q)
                drain_gathers(y_hbm, g, p)
                if do_stage:
                    drain_stage(g, p)
                fire_scatters(p)

            if do_stage:
                stage(0, 0)
            fire_gathers(y_hbm, 0, 0)
            body(0, 0, True, False)

            def pair(k3, carry):
                g0 = k3 * 2 + 1
                body(g0, 1, False, False)
                body(g0 + 1, 0, False, False)
                return carry

            lax.fori_loop(0, (NG - 3) // 2, pair, 0)
            body(NG - 2, (NG - 2) % 2, False, False)
            body(NG - 1, (NG - 1) % 2, False, True)
            drain_scatters((NG - 1) % 2)

            plsc.subcore_barrier()
            pltpu.sync_copy(acc.at[pl.ds(base, RPT)],
                            out_hbm.at[half, c, pl.ds(base, RPT)])
            plsc.subcore_barrier()

    return k


_sc_scatter_h2 = _make_sc_scatter(2)
_sc_scatter_h1 = _make_sc_scatter(1)


def _tc_ab_body(hist_ref, a_ref, b_ref):
    h = hist_ref[...]
    dego = jnp.maximum(h[0, :, 0:1] + h[1, :, 0:1], 1.0)
    degi = jnp.maximum(h[0, :, 1:2] + h[1, :, 1:2], 1.0)
    a_ref[...] = lax.rsqrt(dego)
    b_ref[...] = lax.rsqrt(degi)


def _tc_ab(hist):
    return pl.pallas_call(
        _tc_ab_body,
        grid=(NBLK,),
        in_specs=[pl.BlockSpec((NC, BLK, 16), lambda i: (0, i, 0))],
        out_specs=[
            pl.BlockSpec((BLK, 1), lambda i: (i, 0)),
            pl.BlockSpec((BLK, 1), lambda i: (i, 0)),
        ],
        out_shape=[
            jax.ShapeDtypeStruct((N, 1), jnp.float32),
            jax.ShapeDtypeStruct((N, 1), jnp.float32),
        ],
    )(hist)


def _tc_mm_body(nh, x_ref, a_ref, w0_ref, w1_ref, z_ref, *y_refs):
    x = x_ref[...]
    z_ref[...] = jnp.dot(x, w0_ref[...], preferred_element_type=jnp.float32)
    y = jnp.dot(x * a_ref[...], w1_ref[...],
                preferred_element_type=jnp.float32)
    for h in range(nh):
        y_refs[h][...] = y[:, h * HD:(h + 1) * HD]


def _tc_mm(x, a, w0, w1, nh):
    D = nh * HD
    return pl.pallas_call(
        functools.partial(_tc_mm_body, nh),
        grid=(NBLK,),
        in_specs=[
            pl.BlockSpec((BLK, 128), lambda i: (i, 0)),
            pl.BlockSpec((BLK, 1), lambda i: (i, 0)),
            pl.BlockSpec((128, D), lambda i: (0, 0)),
            pl.BlockSpec((128, D), lambda i: (0, 0)),
        ],
        out_specs=[pl.BlockSpec((BLK, D), lambda i: (i, 0))] +
        [pl.BlockSpec((BLK, HD), lambda i: (i, 0)) for _ in range(nh)],
        out_shape=[jax.ShapeDtypeStruct((N, D), jnp.float32)] +
        [jax.ShapeDtypeStruct((N, HD), jnp.float32) for _ in range(nh)],
    )(x, a, w0, w1)


def _tc_pre_body(nh, z_ref, s_ref, b_ref, sn_ref, bias_ref, pre_ref, st_ref,
                 acc):
    i = pl.program_id(0)

    @pl.when(i == 0)
    def _():
        acc[...] = jnp.zeros_like(acc)

    sv = s_ref[...]
    halves = [sv[h, 0] + sv[h, 1] for h in range(nh)]
    s_full = halves[0] if nh == 1 else jnp.concatenate(halves, axis=-1)
    pre = (z_ref[...] - b_ref[...] * s_full + bias_ref[...])
    pre = pre * sn_ref[...]
    pre_ref[...] = pre
    acc[0:1, :] += jnp.sum(pre, axis=0, keepdims=True)
    acc[1:2, :] += jnp.sum(pre * pre, axis=0, keepdims=True)
    st_ref[...] = acc[...]


def _tc_pre(z, s, b, snorm, bias, nh):
    D = nh * HD
    return pl.pallas_call(
        functools.partial(_tc_pre_body, nh),
        grid=(NBLK,),
        in_specs=[
            pl.BlockSpec((BLK, D), lambda i: (i, 0)),
            pl.BlockSpec((nh, NC, BLK, HD), lambda i: (0, 0, i, 0)),
            pl.BlockSpec((BLK, 1), lambda i: (i, 0)),
            pl.BlockSpec((BLK, 1), lambda i: (i, 0)),
            pl.BlockSpec((1, D), lambda i: (0, 0)),
        ],
        out_specs=[
            pl.BlockSpec((BLK, D), lambda i: (i, 0)),
            pl.BlockSpec((8, D), lambda i: (0, 0)),
        ],
        out_shape=[
            jax.ShapeDtypeStruct((N, D), jnp.float32),
            jax.ShapeDtypeStruct((8, D), jnp.float32),
        ],
        scratch_shapes=[pltpu.VMEM((8, D), jnp.float32)],
    )(z, s, b, snorm, bias)


def _tc_norm_body(pre_ref, st_ref, g_ref, be_ref, xin_ref, o_ref, *, relu,
                  resid):
    st = st_ref[...]
    mean = st[0:1, :] * (1.0 / N)
    var = st[1:2, :] * (1.0 / N) - mean * mean
    inv = lax.rsqrt(var + 1e-5)
    o = (pre_ref[...] - mean) * inv * g_ref[...] + be_ref[...]
    if relu:
        o = jnp.maximum(o, 0.0)
    if resid:
        o = o + xin_ref[...]
    o_ref[...] = o


def _tc_norm(pre, st, gamma, beta, xin, D, relu, resid):
    return pl.pallas_call(
        functools.partial(_tc_norm_body, relu=relu, resid=resid),
        grid=(NBLK,),
        in_specs=[
            pl.BlockSpec((BLK, D), lambda i: (i, 0)),
            pl.BlockSpec((8, D), lambda i: (0, 0)),
            pl.BlockSpec((1, D), lambda i: (0, 0)),
            pl.BlockSpec((1, D), lambda i: (0, 0)),
            pl.BlockSpec((BLK, 128), lambda i: (i, 0)),
        ],
        out_specs=pl.BlockSpec((BLK, D), lambda i: (i, 0)),
        out_shape=jax.ShapeDtypeStruct((N, D), jnp.float32),
    )(pre, st, gamma, beta, xin)


def kernel(h, edge_index, e, snorm_n, snorm_e,
           W0_l0, W1_l0, b_l0, gamma_l0, beta_l0,
           W0_l1, W1_l1, b_l1, gamma_l1, beta_l1,
           W0_l2, W1_l2, b_l2, gamma_l2, beta_l2,
           W0_l3, W1_l3, b_l3, gamma_l3, beta_l3,
           W0_l4, W1_l4, b_l4, gamma_l4, beta_l4):
    srcf = edge_index[0]
    dstf = edge_index[1]

    hist = _sc_degrees(srcf, dstf)
    a, b = _tc_ab(hist)

    layers = [
        (W0_l0, W1_l0, b_l0, gamma_l0, beta_l0),
        (W0_l1, W1_l1, b_l1, gamma_l1, beta_l1),
        (W0_l2, W1_l2, b_l2, gamma_l2, beta_l2),
        (W0_l3, W1_l3, b_l3, gamma_l3, beta_l3),
        (W0_l4, W1_l4, b_l4, gamma_l4, beta_l4),
    ]

    x = h
    for li, (w0, w1, bias, gamma, beta) in enumerate(layers):
        last = li == len(layers) - 1
        nh = 1 if last else 2
        D = nh * HD
        if last:
            w0 = jnp.pad(w0, ((0, 0), (0, D - w0.shape[1])))
            w1 = jnp.pad(w1, ((0, 0), (0, D - w1.shape[1])))
            bias = jnp.pad(bias, (0, D - bias.shape[0]))
            gamma = jnp.pad(gamma, (0, D - gamma.shape[0]))
            beta = jnp.pad(beta, (0, D - beta.shape[0]))
        outs = _tc_mm(x, a, w0, w1, nh)
        z, ys = outs[0], outs[1:]
        if last:
            s = _sc_scatter_h1(ys[0], srcf, dstf)
        else:
            s = _sc_scatter_h2(ys[0], ys[1], srcf, dstf)
        pre, st = _tc_pre(z, s, b, snorm_n, bias.reshape(1, D), nh)
        x = _tc_norm(pre, st, gamma.reshape(1, D), beta.reshape(1, D), x, D,
                     relu=not last, resid=not last)
    return x[:, :40]
